# baseline (speedup 1.0000x reference)
import functools

import jax
import jax.numpy as jnp
from jax import lax
from jax.experimental import pallas as pl
from jax.experimental.pallas import tpu as pltpu
from jax.experimental.pallas import tpu_sc as plsc

N_CELL, N_GENE, N_PEAK = 25000, 15000, 10000
N = N_CELL + N_GENE + N_PEAK
D_IN, D_EMB, H = 512, 256, 64
E = 800000
R = 6
L = 2

CH = 128
E_PAD = 32 * 196 * CH
NROW = E_PAD // CH
T_ROWS = 50048
ROWS_PT = T_ROWS // 16
SB = 128
CHUNKS = tuple((k * SB, SB) for k in range(24)) + ((24 * SB, 56),)
DUMMY = N

BN = 5000
RBN = 5000

BLK = 8
EPT = E_PAD // 16
NBLK = EPT // (BLK * CH)
NBUF = 6


def _encode_body(x_ref, w1_ref, b1_ref, w2_ref, b2_ref, o_ref):
    h = jax.nn.relu(
        jnp.dot(x_ref[...], w1_ref[...], preferred_element_type=jnp.float32)
        + b1_ref[...])
    o_ref[...] = jnp.tanh(
        jnp.dot(h, w2_ref[...], preferred_element_type=jnp.float32)
        + b2_ref[...])


def _encode(feat, w1, b1, w2, b2):
    n = feat.shape[0]
    return pl.pallas_call(
        _encode_body,
        grid=(n // BN,),
        in_specs=[
            pl.BlockSpec((BN, D_IN), lambda i: (i, 0)),
            pl.BlockSpec((D_IN, D_EMB), lambda i: (0, 0)),
            pl.BlockSpec((1, D_EMB), lambda i: (0, 0)),
            pl.BlockSpec((D_EMB, H), lambda i: (0, 0)),
            pl.BlockSpec((1, H), lambda i: (0, 0)),
        ],
        out_specs=pl.BlockSpec((BN, H), lambda i: (i, 0)),
        out_shape=jax.ShapeDtypeStruct((n, H), jnp.float32),
    )(feat, w1, b1.reshape(1, D_EMB), w2, b2.reshape(1, H))


def _relmm_body(x_ref, w_ref, o_ref):
    o_ref[...] = jnp.dot(x_ref[...], w_ref[...],
                         preferred_element_type=jnp.float32)


def _relmm(x, w_cat):
    return pl.pallas_call(
        _relmm_body,
        grid=(N // RBN,),
        in_specs=[
            pl.BlockSpec((RBN, H), lambda i: (i, 0)),
            pl.BlockSpec((H, R * H), lambda i: (0, 0)),
        ],
        out_specs=pl.BlockSpec((RBN, R * H), lambda i: (i, 0)),
        out_shape=jax.ShapeDtypeStruct((N, R * H), jnp.float32),
    )(x, w_cat)


def _ln_update(x, a_ref, d_ref, g_ref, b_ref):
    agg = jnp.concatenate([a_ref[0], a_ref[1]], axis=-1)
    deg = d_ref[0, :, :1] + d_ref[1, :, :1]
    scale = 1.0 / jnp.maximum(deg, 1.0)
    y = x + jax.nn.gelu(agg * scale)
    mu = jnp.mean(y, axis=-1, keepdims=True)
    var = jnp.mean((y - mu) ** 2, axis=-1, keepdims=True)
    return (y - mu) * lax.rsqrt(var + 1e-5) * g_ref[...] + b_ref[...]


def _update_body(x_ref, a_ref, d_ref, g_ref, b_ref, o_ref):
    o_ref[...] = _ln_update(x_ref[...], a_ref, d_ref, g_ref, b_ref)


def _update(x, agg2, deg2, gamma, beta):
    return pl.pallas_call(
        _update_body,
        grid=(N // BN,),
        in_specs=[
            pl.BlockSpec((BN, H), lambda i: (i, 0)),
            pl.BlockSpec((2, BN, 32), lambda i: (0, i, 0)),
            pl.BlockSpec((2, BN, 16), lambda i: (0, i, 0)),
            pl.BlockSpec((1, H), lambda i: (0, 0)),
            pl.BlockSpec((1, H), lambda i: (0, 0)),
        ],
        out_specs=pl.BlockSpec((BN, H), lambda i: (i, 0)),
        out_shape=jax.ShapeDtypeStruct((N, H), jnp.float32),
    )(x, agg2, deg2, gamma.reshape(1, H), beta.reshape(1, H))


@functools.cache
def _sc_agg_call():
    mesh = plsc.VectorSubcoreMesh(
        core_axis_name="c", subcore_axis_name="s",
        num_cores=2, num_subcores=16)
    return pl.kernel(
        _sc_agg_body,
        out_type=jax.ShapeDtypeStruct((2, T_ROWS, 32), jnp.float32),
        mesh=mesh,
        scratch_types=[
            pltpu.VMEM((2, BLK, CH), jnp.int32),
            pltpu.VMEM((2, BLK, CH), jnp.int32),
            pltpu.VMEM((NBUF, CH, 32), jnp.float32),
            pltpu.VMEM_SHARED((T_ROWS, 32), jnp.float32),
            [pltpu.SemaphoreType.DMA] * NBUF,
            [pltpu.SemaphoreType.DMA] * NBUF,
            [pltpu.SemaphoreType.DMA] * 2,
        ],
        compiler_params=pltpu.CompilerParams(use_tc_tiling_on_sc=False),
    )


def _sc_agg_body(xr2_hbm, eidx_a_hbm, eidx_b_hbm, dst_hbm, zeros_hbm,
                 out_hbm, gi, si, rows, table, gsems, ssems, isems):
    c = lax.axis_index("c")
    s = lax.axis_index("s")
    pltpu.sync_copy(zeros_hbm, rows.at[0])
    for off, nr in CHUNKS:
        pltpu.sync_copy(rows.at[0, pl.ds(0, nr)],
                        table.at[pl.ds(s * ROWS_PT + off, nr)])
    plsc.subcore_barrier()

    def load_idx(b, p):
        row0 = s * (EPT // CH) + b * BLK

        @pl.when(c == 0)
        def _():
            pltpu.async_copy(eidx_a_hbm.at[pl.ds(row0, BLK)],
                             gi.at[p], isems[p])

        @pl.when(c == 1)
        def _():
            pltpu.async_copy(eidx_b_hbm.at[pl.ds(row0, BLK)],
                             gi.at[p], isems[p])

        pltpu.async_copy(dst_hbm.at[pl.ds(row0, BLK)], si.at[p], isems[p])

    def wait_idx(p):
        pltpu.make_async_copy(dst_hbm.at[pl.ds(0, BLK)],
                              gi.at[p], isems[p]).wait()
        pltpu.make_async_copy(dst_hbm.at[pl.ds(0, BLK)],
                              si.at[p], isems[p]).wait()

    def do_block(p):
        for k in range(5):
            b = k % NBUF
            pltpu.async_copy(xr2_hbm.at[gi.at[p, k]], rows.at[b], gsems[b])
        for k in range(BLK):
            b = k % NBUF
            pltpu.make_async_copy(xr2_hbm.at[gi.at[p, k]], rows.at[b],
                                  gsems[b]).wait()
            pltpu.async_copy(rows.at[b], table.at[si.at[p, k]],
                             ssems[b], add=True)
            if k + 5 < BLK:
                b5 = (k + 5) % NBUF
                if k - 1 >= 0:
                    pltpu.make_async_copy(
                        rows.at[b5], table.at[si.at[p, k - 1]],
                        ssems[b5]).wait()
                pltpu.async_copy(xr2_hbm.at[gi.at[p, k + 5]],
                                 rows.at[b5], gsems[b5])
        for k in range(2, BLK):
            b = k % NBUF
            pltpu.make_async_copy(rows.at[b], table.at[si.at[p, k]],
                                  ssems[b]).wait()

    load_idx(0, 0)

    def outer(j, carry):
        wait_idx(0)
        load_idx(2 * j + 1, 1)
        do_block(0)
        wait_idx(1)
        load_idx(2 * j + 2, 0)
        do_block(1)
        return carry

    lax.fori_loop(0, NBLK // 2, outer, 0)
    wait_idx(0)
    do_block(0)
    plsc.subcore_barrier()

    for m, (off, nr) in enumerate(CHUNKS):
        b = m % NBUF
        r0 = s * ROWS_PT + off
        if m >= NBUF:
            off_p, nr_p = CHUNKS[m - NBUF]
            pltpu.make_async_copy(
                rows.at[b, pl.ds(0, nr_p)],
                out_hbm.at[c, pl.ds(s * ROWS_PT + off_p, nr_p)],
                gsems[b]).wait()
        pltpu.sync_copy(table.at[pl.ds(r0, nr)], rows.at[b, pl.ds(0, nr)])
        pltpu.async_copy(rows.at[b, pl.ds(0, nr)],
                         out_hbm.at[c, pl.ds(r0, nr)], gsems[b])
    for m in range(len(CHUNKS) - NBUF, len(CHUNKS)):
        b = m % NBUF
        off, nr = CHUNKS[m]
        pltpu.make_async_copy(
            rows.at[b, pl.ds(0, nr)],
            out_hbm.at[c, pl.ds(s * ROWS_PT + off, nr)], gsems[b]).wait()


DBLK = 8
DMAIN = 24


@functools.cache
def _sc_deg_call():
    mesh = plsc.VectorSubcoreMesh(
        core_axis_name="c", subcore_axis_name="s",
        num_cores=2, num_subcores=16)
    return pl.kernel(
        _sc_deg_body,
        out_type=jax.ShapeDtypeStruct((2, T_ROWS, 16), jnp.float32),
        mesh=mesh,
        scratch_types=[
            pltpu.VMEM((2, DBLK, CH), jnp.int32),
            pltpu.VMEM((CH, 16), jnp.float32),
            pltpu.VMEM((SB, 16), jnp.float32),
            pltpu.VMEM_SHARED((T_ROWS, 16), jnp.float32),
            [pltpu.SemaphoreType.DMA] * 2,
        ],
        compiler_params=pltpu.CompilerParams(use_tc_tiling_on_sc=False),
    )


def _sc_deg_body(dst_hbm, ones_hbm, zeros_hbm, out_hbm,
                 si, ones_v, stage, table, isems):
    c = lax.axis_index("c")
    s = lax.axis_index("s")
    w = c * 16 + s
    pltpu.sync_copy(zeros_hbm, stage)
    for off, nr in CHUNKS:
        pltpu.sync_copy(stage.at[pl.ds(0, nr)],
                        table.at[pl.ds(s * ROWS_PT + off, nr)])
    pltpu.sync_copy(ones_hbm, ones_v)
    plsc.subcore_barrier()

    def load_idx(row0, p):
        pltpu.async_copy(dst_hbm.at[pl.ds(row0, DBLK)], si.at[p], isems[p])

    def wait_idx(p):
        pltpu.make_async_copy(dst_hbm.at[pl.ds(0, DBLK)],
                              si.at[p], isems[p]).wait()

    def do_block(p):
        for k in range(DBLK):
            pltpu.sync_copy(ones_v, table.at[si.at[p, k]], add=True)

    base = w * (DMAIN * DBLK)
    load_idx(base, 0)

    def outer(j, carry):
        wait_idx(0)
        load_idx(base + (2 * j + 1) * DBLK, 1)
        do_block(0)
        wait_idx(1)

        @pl.when(j < DMAIN // 2 - 1)
        def _():
            load_idx(base + (2 * j + 2) * DBLK, 0)

        @pl.when(jnp.logical_and(j == DMAIN // 2 - 1, w < 16))
        def _():
            load_idx(32 * DMAIN * DBLK + w * DBLK, 0)

        do_block(1)
        return carry

    lax.fori_loop(0, DMAIN // 2, outer, 0)

    @pl.when(w < 16)
    def _():
        wait_idx(0)
        do_block(0)

    plsc.subcore_barrier()

    for off, nr in CHUNKS:
        r0 = s * ROWS_PT + off
        pltpu.sync_copy(table.at[pl.ds(r0, nr)], stage.at[pl.ds(0, nr)])
        pltpu.sync_copy(stage.at[pl.ds(0, nr)], out_hbm.at[c, pl.ds(r0, nr)])


def kernel(cell_feature, gene_feature, peak_feature, node_type, edge_index,
           edge_type, W_emb, b_emb, W_adapt, b_adapt, W_rel, ln_gamma,
           ln_beta):
    del node_type
    src = edge_index[0]
    dst = edge_index[1]

    base2 = 2 * (src * R + edge_type)
    eidx_a = jnp.pad(base2, (0, E_PAD - E)).reshape(-1, CH)
    eidx_b = jnp.pad(base2 + 1, (0, E_PAD - E)).reshape(-1, CH)
    dst2 = jnp.pad(dst, (0, E_PAD - E),
                   constant_values=DUMMY).reshape(-1, CH)
    zeros32 = jnp.zeros((SB, 32), jnp.float32)
    zeros16 = jnp.zeros((SB, 16), jnp.float32)
    ones16 = jnp.ones((CH, 16), jnp.float32)

    x = jnp.concatenate([
        _encode(cell_feature, W_emb[0], b_emb[0], W_adapt[0], b_adapt[0]),
        _encode(gene_feature, W_emb[1], b_emb[1], W_adapt[1], b_adapt[1]),
        _encode(peak_feature, W_emb[2], b_emb[2], W_adapt[2], b_adapt[2]),
    ], axis=0)

    deg2 = _sc_deg_call()(dst2, ones16, zeros16)

    for l in range(L):
        w_cat = jnp.transpose(W_rel[l], (1, 0, 2)).reshape(H, R * H)
        xr = _relmm(x, w_cat)
        agg2 = _sc_agg_call()(xr.reshape(N * R * 2, 32), eidx_a, eidx_b,
                              dst2, zeros32)
        x = _update(x, agg2, deg2, ln_gamma[l], ln_beta[l])
    return x


# speedup vs baseline: 1.0119x; 1.0119x over previous
import functools

import jax
import jax.numpy as jnp
from jax import lax
from jax.experimental import pallas as pl
from jax.experimental.pallas import tpu as pltpu
from jax.experimental.pallas import tpu_sc as plsc

N_CELL, N_GENE, N_PEAK = 25000, 15000, 10000
N = N_CELL + N_GENE + N_PEAK
D_IN, D_EMB, H = 512, 256, 64
E = 800000
R = 6
L = 2

CH = 128
E_PAD = 32 * 196 * CH
NROW = E_PAD // CH
T_ROWS = 50048
ROWS_PT = T_ROWS // 16
SB = 128
CHUNKS = tuple((k * SB, SB) for k in range(24)) + ((24 * SB, 56),)
DUMMY = N

BN = 5000
RBN = 5000

BLK = 8
EPT = E_PAD // 16
NBLK = EPT // (BLK * CH)
NBUF = 6


def _encode_body(x_ref, w1_ref, b1_ref, w2_ref, b2_ref, o_ref):
    h = jax.nn.relu(
        jnp.dot(x_ref[...], w1_ref[...], preferred_element_type=jnp.float32)
        + b1_ref[...])
    o_ref[...] = jnp.tanh(
        jnp.dot(h, w2_ref[...], preferred_element_type=jnp.float32)
        + b2_ref[...])


def _encode(feat, w1, b1, w2, b2):
    n = feat.shape[0]
    return pl.pallas_call(
        _encode_body,
        grid=(n // BN,),
        in_specs=[
            pl.BlockSpec((BN, D_IN), lambda i: (i, 0)),
            pl.BlockSpec((D_IN, D_EMB), lambda i: (0, 0)),
            pl.BlockSpec((1, D_EMB), lambda i: (0, 0)),
            pl.BlockSpec((D_EMB, H), lambda i: (0, 0)),
            pl.BlockSpec((1, H), lambda i: (0, 0)),
        ],
        out_specs=pl.BlockSpec((BN, H), lambda i: (i, 0)),
        out_shape=jax.ShapeDtypeStruct((n, H), jnp.float32),
    )(feat, w1, b1.reshape(1, D_EMB), w2, b2.reshape(1, H))


def _relmm_body(x_ref, w_ref, o_ref):
    o_ref[...] = jnp.dot(x_ref[...], w_ref[...],
                         preferred_element_type=jnp.float32)


def _relmm(x, w_cat):
    return pl.pallas_call(
        _relmm_body,
        grid=(N // RBN,),
        in_specs=[
            pl.BlockSpec((RBN, H), lambda i: (i, 0)),
            pl.BlockSpec((H, R * H), lambda i: (0, 0)),
        ],
        out_specs=pl.BlockSpec((RBN, R * H), lambda i: (i, 0)),
        out_shape=jax.ShapeDtypeStruct((N, R * H), jnp.float32),
    )(x, w_cat)


def _ln_update(x, a_ref, d_ref, g_ref, b_ref):
    agg = jnp.concatenate([a_ref[0], a_ref[1]], axis=-1)
    deg = d_ref[0, :, :1] + d_ref[1, :, :1]
    scale = 1.0 / jnp.maximum(deg, 1.0)
    y = x + jax.nn.gelu(agg * scale)
    mu = jnp.mean(y, axis=-1, keepdims=True)
    var = jnp.mean((y - mu) ** 2, axis=-1, keepdims=True)
    return (y - mu) * lax.rsqrt(var + 1e-5) * g_ref[...] + b_ref[...]


def _update_body(x_ref, a_ref, d_ref, g_ref, b_ref, o_ref):
    o_ref[...] = _ln_update(x_ref[...], a_ref, d_ref, g_ref, b_ref)


def _update(x, agg2, deg2, gamma, beta):
    return pl.pallas_call(
        _update_body,
        grid=(N // BN,),
        in_specs=[
            pl.BlockSpec((BN, H), lambda i: (i, 0)),
            pl.BlockSpec((2, BN, 32), lambda i: (0, i, 0)),
            pl.BlockSpec((2, BN, 16), lambda i: (0, i, 0)),
            pl.BlockSpec((1, H), lambda i: (0, 0)),
            pl.BlockSpec((1, H), lambda i: (0, 0)),
        ],
        out_specs=pl.BlockSpec((BN, H), lambda i: (i, 0)),
        out_shape=jax.ShapeDtypeStruct((N, H), jnp.float32),
    )(x, agg2, deg2, gamma.reshape(1, H), beta.reshape(1, H))


@functools.cache
def _sc_agg_call():
    mesh = plsc.VectorSubcoreMesh(
        core_axis_name="c", subcore_axis_name="s",
        num_cores=2, num_subcores=16)
    return pl.kernel(
        _sc_agg_body,
        out_type=jax.ShapeDtypeStruct((2, T_ROWS, 32), jnp.float32),
        mesh=mesh,
        scratch_types=[
            pltpu.VMEM((2, BLK, CH), jnp.int32),
            pltpu.VMEM((2, BLK, CH), jnp.int32),
            pltpu.VMEM((NBUF, CH, 32), jnp.float32),
            pltpu.VMEM_SHARED((T_ROWS, 32), jnp.float32),
            [pltpu.SemaphoreType.DMA] * NBUF,
            [pltpu.SemaphoreType.DMA] * NBUF,
            [pltpu.SemaphoreType.DMA] * 2,
        ],
        compiler_params=pltpu.CompilerParams(use_tc_tiling_on_sc=False),
    )


def _sc_agg_body(xr2_hbm, eidx_a_hbm, eidx_b_hbm, dst_hbm, zeros_hbm,
                 out_hbm, gi, si, rows, table, gsems, ssems, isems):
    c = lax.axis_index("c")
    s = lax.axis_index("s")
    pltpu.sync_copy(zeros_hbm, rows.at[0])
    for off, nr in CHUNKS:
        pltpu.sync_copy(rows.at[0, pl.ds(0, nr)],
                        table.at[pl.ds(s * ROWS_PT + off, nr)])
    plsc.subcore_barrier()

    def load_idx(b, p):
        row0 = s * (EPT // CH) + b * BLK

        @pl.when(c == 0)
        def _():
            pltpu.async_copy(eidx_a_hbm.at[pl.ds(row0, BLK)],
                             gi.at[p], isems[p])

        @pl.when(c == 1)
        def _():
            pltpu.async_copy(eidx_b_hbm.at[pl.ds(row0, BLK)],
                             gi.at[p], isems[p])

        pltpu.async_copy(dst_hbm.at[pl.ds(row0, BLK)], si.at[p], isems[p])

    def wait_idx(p):
        pltpu.make_async_copy(dst_hbm.at[pl.ds(0, BLK)],
                              gi.at[p], isems[p]).wait()
        pltpu.make_async_copy(dst_hbm.at[pl.ds(0, BLK)],
                              si.at[p], isems[p]).wait()

    def do_block(p):
        for k in range(4):
            b = k % NBUF
            pltpu.async_copy(xr2_hbm.at[gi.at[p, k]], rows.at[b], gsems[b])
        for k in range(BLK):
            b = k % NBUF
            pltpu.make_async_copy(xr2_hbm.at[gi.at[p, k]], rows.at[b],
                                  gsems[b]).wait()
            pltpu.async_copy(rows.at[b], table.at[si.at[p, k]],
                             ssems[b], add=True)
            if k + 4 < BLK:
                b4 = (k + 4) % NBUF
                if k - 2 >= 0:
                    pltpu.make_async_copy(
                        rows.at[b4], table.at[si.at[p, k - 2]],
                        ssems[b4]).wait()
                pltpu.async_copy(xr2_hbm.at[gi.at[p, k + 4]],
                                 rows.at[b4], gsems[b4])
        for k in range(2, BLK):
            b = k % NBUF
            pltpu.make_async_copy(rows.at[b], table.at[si.at[p, k]],
                                  ssems[b]).wait()

    load_idx(0, 0)

    def outer(j, carry):
        wait_idx(0)
        load_idx(2 * j + 1, 1)
        do_block(0)
        wait_idx(1)
        load_idx(2 * j + 2, 0)
        do_block(1)
        return carry

    lax.fori_loop(0, NBLK // 2, outer, 0)
    wait_idx(0)
    do_block(0)
    plsc.subcore_barrier()

    for m, (off, nr) in enumerate(CHUNKS):
        b = m % NBUF
        r0 = s * ROWS_PT + off
        if m >= NBUF:
            off_p, nr_p = CHUNKS[m - NBUF]
            pltpu.make_async_copy(
                rows.at[b, pl.ds(0, nr_p)],
                out_hbm.at[c, pl.ds(s * ROWS_PT + off_p, nr_p)],
                gsems[b]).wait()
        pltpu.sync_copy(table.at[pl.ds(r0, nr)], rows.at[b, pl.ds(0, nr)])
        pltpu.async_copy(rows.at[b, pl.ds(0, nr)],
                         out_hbm.at[c, pl.ds(r0, nr)], gsems[b])
    for m in range(len(CHUNKS) - NBUF, len(CHUNKS)):
        b = m % NBUF
        off, nr = CHUNKS[m]
        pltpu.make_async_copy(
            rows.at[b, pl.ds(0, nr)],
            out_hbm.at[c, pl.ds(s * ROWS_PT + off, nr)], gsems[b]).wait()


DBLK = 8
DMAIN = 24


@functools.cache
def _sc_deg_call():
    mesh = plsc.VectorSubcoreMesh(
        core_axis_name="c", subcore_axis_name="s",
        num_cores=2, num_subcores=16)
    return pl.kernel(
        _sc_deg_body,
        out_type=jax.ShapeDtypeStruct((2, T_ROWS, 16), jnp.float32),
        mesh=mesh,
        scratch_types=[
            pltpu.VMEM((2, DBLK, CH), jnp.int32),
            pltpu.VMEM((CH, 16), jnp.float32),
            pltpu.VMEM((SB, 16), jnp.float32),
            pltpu.VMEM_SHARED((T_ROWS, 16), jnp.float32),
            [pltpu.SemaphoreType.DMA] * 2,
        ],
        compiler_params=pltpu.CompilerParams(use_tc_tiling_on_sc=False),
    )


def _sc_deg_body(dst_hbm, ones_hbm, zeros_hbm, out_hbm,
                 si, ones_v, stage, table, isems):
    c = lax.axis_index("c")
    s = lax.axis_index("s")
    w = c * 16 + s
    pltpu.sync_copy(zeros_hbm, stage)
    for off, nr in CHUNKS:
        pltpu.sync_copy(stage.at[pl.ds(0, nr)],
                        table.at[pl.ds(s * ROWS_PT + off, nr)])
    pltpu.sync_copy(ones_hbm, ones_v)
    plsc.subcore_barrier()

    def load_idx(row0, p):
        pltpu.async_copy(dst_hbm.at[pl.ds(row0, DBLK)], si.at[p], isems[p])

    def wait_idx(p):
        pltpu.make_async_copy(dst_hbm.at[pl.ds(0, DBLK)],
                              si.at[p], isems[p]).wait()

    def do_block(p):
        for k in range(DBLK):
            pltpu.sync_copy(ones_v, table.at[si.at[p, k]], add=True)

    base = w * (DMAIN * DBLK)
    load_idx(base, 0)

    def outer(j, carry):
        wait_idx(0)
        load_idx(base + (2 * j + 1) * DBLK, 1)
        do_block(0)
        wait_idx(1)

        @pl.when(j < DMAIN // 2 - 1)
        def _():
            load_idx(base + (2 * j + 2) * DBLK, 0)

        @pl.when(jnp.logical_and(j == DMAIN // 2 - 1, w < 16))
        def _():
            load_idx(32 * DMAIN * DBLK + w * DBLK, 0)

        do_block(1)
        return carry

    lax.fori_loop(0, DMAIN // 2, outer, 0)

    @pl.when(w < 16)
    def _():
        wait_idx(0)
        do_block(0)

    plsc.subcore_barrier()

    for off, nr in CHUNKS:
        r0 = s * ROWS_PT + off
        pltpu.sync_copy(table.at[pl.ds(r0, nr)], stage.at[pl.ds(0, nr)])
        pltpu.sync_copy(stage.at[pl.ds(0, nr)], out_hbm.at[c, pl.ds(r0, nr)])


def kernel(cell_feature, gene_feature, peak_feature, node_type, edge_index,
           edge_type, W_emb, b_emb, W_adapt, b_adapt, W_rel, ln_gamma,
           ln_beta):
    del node_type
    src = edge_index[0]
    dst = edge_index[1]

    base2 = 2 * (src * R + edge_type)
    eidx_a = jnp.pad(base2, (0, E_PAD - E)).reshape(-1, CH)
    eidx_b = jnp.pad(base2 + 1, (0, E_PAD - E)).reshape(-1, CH)
    dst2 = jnp.pad(dst, (0, E_PAD - E),
                   constant_values=DUMMY).reshape(-1, CH)
    zeros32 = jnp.zeros((SB, 32), jnp.float32)
    zeros16 = jnp.zeros((SB, 16), jnp.float32)
    ones16 = jnp.ones((CH, 16), jnp.float32)

    x = jnp.concatenate([
        _encode(cell_feature, W_emb[0], b_emb[0], W_adapt[0], b_adapt[0]),
        _encode(gene_feature, W_emb[1], b_emb[1], W_adapt[1], b_adapt[1]),
        _encode(peak_feature, W_emb[2], b_emb[2], W_adapt[2], b_adapt[2]),
    ], axis=0)

    deg2 = _sc_deg_call()(dst2, ones16, zeros16)

    for l in range(L):
        w_cat = jnp.transpose(W_rel[l], (1, 0, 2)).reshape(H, R * H)
        xr = _relmm(x, w_cat)
        agg2 = _sc_agg_call()(xr.reshape(N * R * 2, 32), eidx_a, eidx_b,
                              dst2, zeros32)
        x = _update(x, agg2, deg2, ln_gamma[l], ln_beta[l])
    return x
